# baseline (speedup 1.0000x reference)
import functools

import jax
import jax.numpy as jnp
from jax import lax
from jax.experimental import pallas as pl
from jax.experimental.pallas import tpu as pltpu
from jax.experimental.pallas import tpu_sc as plsc

VOCAB = 400000
EMB_DIM = 100
HIDDEN = 128
NUM_CLASSES = 4
BATCH = 4096
SEQLEN = 200

DP = 112
NC = 2
NS = 16
NW = NC * NS
CHUNK = 100
CP = 112
CPW = (BATCH * SEQLEN) // (NW * CHUNK)
RPW = BATCH // NW
LANES = 16
OUT_OFFS = (0, 16, 32, 48, 64, 80, 84)


NVEC = CP // LANES


def _pool_body(x_hbm, tab_hbm, pooled_hbm, idx_v, buf0, buf1, acc, out_v,
               sem0, sem1):
    cid = lax.axis_index("c")
    sid = lax.axis_index("s")
    wid = sid * NC + cid
    cbase = wid * CPW

    bufs = (buf0, buf1)
    sems = (sem0, sem1)

    pltpu.sync_copy(x_hbm.at[pl.ds(cbase, CPW)], idx_v)

    zvec = jnp.zeros((LANES,), jnp.float32)

    def fire(c, k):
        for j in range(NVEC):
            iv = idx_v[c, pl.ds(j * LANES, LANES)]
            pltpu.async_copy(tab_hbm.at[iv],
                             bufs[k].at[pl.ds(j * LANES, LANES)], sems[k])

    def wait_all(c, k):
        pltpu.make_async_copy(tab_hbm.at[idx_v.at[c]], bufs[k],
                              sems[k]).wait()

    def accumulate(buf, carry_in):
        @pl.loop(0, CHUNK // 4, init_carry=carry_in)
        def carry_out(g, carry):
            vs = list(carry)
            for rr in range(4):
                r = g * 4 + rr
                for v in range(DP // LANES):
                    vs[v] = vs[v] + buf[r, pl.ds(v * LANES, LANES)]
            return tuple(vs)

        return carry_out

    fire(0, 0)
    fire(1, 1)

    def do_chunk(c, k, carry):
        wait_all(c, k)
        carry = accumulate(bufs[k], carry)
        fire(jnp.minimum(c + 2, CPW - 2 + k), k)
        return carry

    def store_row(i, acc7):
        for v in range(6):
            out_v[i, pl.ds(v * LANES, LANES)] = acc7[v]
        acc[pl.ds(80, LANES)] = acc7[5]
        acc[pl.ds(96, LANES)] = acc7[6]
        out_v[i, pl.ds(84, LANES)] = acc[pl.ds(84, LANES)]

    @pl.loop(0, RPW)
    def _(i):
        acc7 = (zvec,) * (DP // LANES)
        acc7 = do_chunk(2 * i, 0, acc7)
        acc7 = do_chunk(2 * i + 1, 1, acc7)
        store_row(i, acc7)

    wait_all(CPW - 2, 0)
    wait_all(CPW - 1, 1)

    pltpu.sync_copy(out_v, pooled_hbm.at[pl.ds(wid * RPW, RPW)])


@functools.partial(
    pl.kernel,
    out_type=jax.ShapeDtypeStruct((BATCH, EMB_DIM), jnp.float32),
    mesh=plsc.VectorSubcoreMesh(core_axis_name="c", subcore_axis_name="s"),
    compiler_params=pltpu.CompilerParams(use_tc_tiling_on_sc=False),
    scratch_types=[
        pltpu.VMEM((CPW, CP), jnp.int32),
        pltpu.VMEM((CP, DP), jnp.float32),
        pltpu.VMEM((CP, DP), jnp.float32),
        pltpu.VMEM((DP,), jnp.float32),
        pltpu.VMEM((RPW, EMB_DIM), jnp.float32),
        pltpu.SemaphoreType.DMA,
        pltpu.SemaphoreType.DMA,
    ],
)
def _pool(x_hbm, tab_hbm, pooled_hbm, *rest):
    _pool_body(x_hbm, tab_hbm, pooled_hbm, *rest)


def _mlp_body(p_ref, w1_ref, b1_ref, w2_ref, b2_ref, o_ref):
    h = jnp.dot(p_ref[...], w1_ref[...], preferred_element_type=jnp.float32)
    h = h * (1.0 / SEQLEN) + b1_ref[...]
    h = jnp.maximum(h, 0.0)
    o_ref[...] = (
        jnp.dot(h, w2_ref[...], preferred_element_type=jnp.float32)
        + b2_ref[...]
    )


_mlp = pl.pallas_call(
    _mlp_body,
    out_shape=jax.ShapeDtypeStruct((BATCH, NUM_CLASSES), jnp.float32),
)


@jax.jit
def kernel(x, emb_table, W1, b1, W2, b2):
    tabp = jnp.pad(emb_table, ((0, 0), (0, DP - EMB_DIM)))
    xp = jnp.pad(x.reshape(-1, CHUNK), ((0, 0), (0, CP - CHUNK)))
    pooled = _pool(xp, tabp)
    return _mlp(pooled, W1, b1.reshape(1, HIDDEN), W2,
                b2.reshape(1, NUM_CLASSES))


# speedup vs baseline: 3.0317x; 3.0317x over previous
import functools

import jax
import jax.numpy as jnp
from jax import lax
from jax.experimental import pallas as pl
from jax.experimental.pallas import tpu as pltpu
from jax.experimental.pallas import tpu_sc as plsc

VOCAB = 400000
EMB_DIM = 100
HIDDEN = 128
NUM_CLASSES = 4
BATCH = 4096
SEQLEN = 200

DP = 128
NC = 2
NS = 16
NW = NC * NS
CHUNK = 100
CP = 104
CPW = (BATCH * SEQLEN) // (NW * CHUNK)
RPW = BATCH // NW
LANES = 16
NBLK = DP // 32
OUT_OFFS = (0, 16, 32, 48, 64, 80, 84)


def _pool_body(x_hbm, tab_hbm, pooled_hbm, idx_v, buf0, buf1, acc, out_v,
               sem0, sem1):
    cid = lax.axis_index("c")
    sid = lax.axis_index("s")
    wid = sid * NC + cid
    cbase = wid * CPW

    bufs = (buf0, buf1)
    sems = (sem0, sem1)

    pltpu.sync_copy(x_hbm.at[pl.ds(cbase, CPW)], idx_v)

    zvec = jnp.zeros((LANES,), jnp.float32)
    himask = jnp.full((LANES,), -65536, jnp.int32)

    def fire(c, k):
        pltpu.async_copy(tab_hbm.at[idx_v.at[c]], bufs[k], sems[k])

    def wait_all(c, k):
        pltpu.make_async_copy(tab_hbm.at[idx_v.at[c]], bufs[k],
                              sems[k]).wait()

    def accumulate(buf, carry_in):
        @pl.loop(0, CHUNK // 2, init_carry=carry_in)
        def carry_out(g, carry):
            vs = list(carry)
            for rr in range(2):
                r = g * 2 + rr
                for b in range(NBLK):
                    w = plsc.bitcast(buf[r, pl.ds(b * 32, 32)], jnp.int32)
                    lo = plsc.bitcast(w << 16, jnp.float32)
                    hi = plsc.bitcast(w & himask, jnp.float32)
                    vs[2 * b] = vs[2 * b] + lo
                    vs[2 * b + 1] = vs[2 * b + 1] + hi
            return tuple(vs)

        return carry_out

    fire(0, 0)
    fire(1, 1)

    def do_chunk(c, k, carry):
        wait_all(c, k)
        carry = accumulate(bufs[k], carry)
        fire(jnp.minimum(c + 2, CPW - 2 + k), k)
        return carry

    evens = lax.iota(jnp.int32, LANES) * 2
    odds = evens + 1

    def store_row(i, acc8):
        for b in range(NBLK):
            base = 32 * b
            plsc.store_scatter(acc, [evens + base], acc8[2 * b])
            plsc.store_scatter(acc, [odds + base], acc8[2 * b + 1])
        for off in OUT_OFFS:
            out_v[i, pl.ds(off, LANES)] = acc[pl.ds(off, LANES)]

    @pl.loop(0, RPW)
    def _(i):
        acc8 = (zvec,) * (2 * NBLK)
        acc8 = do_chunk(2 * i, 0, acc8)
        acc8 = do_chunk(2 * i + 1, 1, acc8)
        store_row(i, acc8)

    wait_all(CPW - 2, 0)
    wait_all(CPW - 1, 1)

    pltpu.sync_copy(out_v, pooled_hbm.at[pl.ds(wid * RPW, RPW)])


@functools.partial(
    pl.kernel,
    out_type=jax.ShapeDtypeStruct((BATCH, EMB_DIM), jnp.float32),
    mesh=plsc.VectorSubcoreMesh(core_axis_name="c", subcore_axis_name="s"),
    compiler_params=pltpu.CompilerParams(use_tc_tiling_on_sc=False,
                                         needs_layout_passes=False),
    scratch_types=[
        pltpu.VMEM((CPW, CP), jnp.int32),
        pltpu.VMEM((CP, DP), jnp.bfloat16),
        pltpu.VMEM((CP, DP), jnp.bfloat16),
        pltpu.VMEM((DP,), jnp.float32),
        pltpu.VMEM((RPW, EMB_DIM), jnp.float32),
        pltpu.SemaphoreType.DMA,
        pltpu.SemaphoreType.DMA,
    ],
)
def _pool(x_hbm, tab_hbm, pooled_hbm, *rest):
    _pool_body(x_hbm, tab_hbm, pooled_hbm, *rest)


def _mlp_body(p_ref, w1_ref, b1_ref, w2_ref, b2_ref, o_ref):
    h = jnp.dot(p_ref[...], w1_ref[...], preferred_element_type=jnp.float32)
    h = h * (1.0 / SEQLEN) + b1_ref[...]
    h = jnp.maximum(h, 0.0)
    o_ref[...] = (
        jnp.dot(h, w2_ref[...], preferred_element_type=jnp.float32)
        + b2_ref[...]
    )


_mlp = pl.pallas_call(
    _mlp_body,
    out_shape=jax.ShapeDtypeStruct((BATCH, NUM_CLASSES), jnp.float32),
)


@jax.jit
def kernel(x, emb_table, W1, b1, W2, b2):
    tabp = jnp.pad(emb_table.astype(jnp.bfloat16), ((0, 0), (0, DP - EMB_DIM)))
    xp = jnp.pad(x.reshape(-1, CHUNK), ((0, 0), (0, CP - CHUNK)))
    pooled = _pool(xp, tabp)
    return _mlp(pooled, W1, b1.reshape(1, HIDDEN), W2,
                b2.reshape(1, NUM_CLASSES))
